# Initial kernel scaffold; baseline (speedup 1.0000x reference)
import jax
import jax.numpy as jnp
from jax.experimental import pallas as pl


def kernel(x, edge_index, batch, W, b, eps, gamma, beta):
    raise NotImplementedError("write your pallas kernel here")


# speedup vs baseline: 4.4078x; 4.4078x over previous
import functools

import jax
import jax.numpy as jnp
from jax import lax
from jax.experimental import pallas as pl
from jax.experimental.pallas import tpu as pltpu
from jax.experimental.pallas import tpu_sc as plsc

NC = 2
NS = 16
LANES = 16

CH = 2560
NCHUNK_PER_SC = 2
NPAD = CH * NC * NCHUNK_PER_SC
ZR = 4
SECT = 2000
B = 32


def _sc_agg(x, src, dst):
    n, d = x.shape
    e = src.shape[0]
    e_per_sub = e // NS
    n_sect = e_per_sub // SECT
    sel = SECT + 2 * B
    rows_per_sub = CH // NS

    mesh = plsc.VectorSubcoreMesh(core_axis_name="c", subcore_axis_name="s")

    @functools.partial(
        pl.kernel,
        mesh=mesh,
        out_type=jax.ShapeDtypeStruct((NPAD, d), jnp.float32),
        compiler_params=pltpu.CompilerParams(
            needs_layout_passes=False, use_tc_tiling_on_sc=False),
        scratch_types=[
            pltpu.VMEM((SECT,), jnp.int32),
            pltpu.VMEM((SECT,), jnp.int32),
            pltpu.VMEM((SECT,), jnp.int32),
            pltpu.VMEM((SECT,), jnp.int32),
            pltpu.VMEM((sel,), jnp.int32),
            pltpu.VMEM((sel,), jnp.int32),
            pltpu.VMEM((B,), jnp.int32),
            pltpu.VMEM((B,), jnp.int32),
            pltpu.VMEM((B, d), jnp.float32),
            pltpu.VMEM((B, d), jnp.float32),
            pltpu.VMEM_SHARED((CH + 8, d), jnp.float32),
            pltpu.SemaphoreType.DMA,
            pltpu.SemaphoreType.DMA,
            pltpu.SemaphoreType.DMA,
            pltpu.SemaphoreType.DMA,
        ],
    )
    def k(x_hbm, src_hbm, dst_hbm, agg_hbm,
          src_v0, dst_v0, src_v1, dst_v1, sel_s, sel_d, stage_a, stage_b,
          rows_a, rows_b, acc, sem_a, sem_b, sem_l0, sem_l1):
        c = lax.axis_index("c")
        s = lax.axis_index("s")
        e0 = s * e_per_sub
        zv = jnp.zeros((LANES,), jnp.float32)
        lane = lax.iota(jnp.int32, LANES)

        def start_load(si, sb, db, sm):
            pltpu.make_async_copy(
                src_hbm.at[pl.ds(e0 + si * SECT, SECT)], sb, sm).start()
            pltpu.make_async_copy(
                dst_hbm.at[pl.ds(e0 + si * SECT, SECT)], db, sm).start()

        def wait_load(sb, db, sm):
            pltpu.make_async_copy(src_hbm.at[pl.ds(e0, SECT)], sb, sm).wait()
            pltpu.make_async_copy(dst_hbm.at[pl.ds(e0, SECT)], db, sm).wait()

        def start_gather(j, rbuf, sm):
            pltpu.make_async_copy(
                x_hbm.at[sel_s.at[pl.ds(j * B, B)]], rbuf, sm).start()

        def drain_gather(rbuf, sm):
            pltpu.make_async_copy(x_hbm.at[sel_s.at[pl.ds(0, B)]], rbuf, sm).wait()

        def stage_scatter(j, stg):
            for kk in range(B // LANES):
                stg[pl.ds(kk * LANES, LANES)] = sel_d[pl.ds(j * B + kk * LANES, LANES)]

        def flush_pipelined(nb):
            @pl.when(nb > 0)
            def _():
                start_gather(0, rows_a, sem_a)

            def pair(i, _):
                j = 2 * i

                @pl.when(j + 1 < nb)
                def _():
                    start_gather(j + 1, rows_b, sem_b)
                stage_scatter(j, stage_a)
                drain_gather(rows_a, sem_a)
                pltpu.sync_copy(rows_a, acc.at[stage_a], add=True)

                @pl.when(j + 1 < nb)
                def _():
                    @pl.when(j + 2 < nb)
                    def _():
                        start_gather(j + 2, rows_a, sem_a)
                    stage_scatter(j + 1, stage_b)
                    drain_gather(rows_b, sem_b)
                    pltpu.sync_copy(rows_b, acc.at[stage_b], add=True)
                return 0
            lax.fori_loop(0, (nb + 1) // 2, pair, 0)

        def do_chunk(cc, _):
            chunk = c * NCHUNK_PER_SC + cc
            lo = chunk * CH

            def zr(r, _):
                def zc(j, _):
                    rows_a[r, pl.ds(j * LANES, LANES)] = zv
                    return 0
                return lax.fori_loop(0, d // LANES, zc, 0)
            lax.fori_loop(0, B, zr, 0)
            for zi in range(rows_per_sub // B):
                pltpu.sync_copy(rows_a, acc.at[pl.ds(s * rows_per_sub + zi * B, B)])
            plsc.subcore_barrier()

            start_load(0, src_v0, dst_v0, sem_l0)
            cnt = jnp.int32(0)
            for si in range(n_sect):
                if si % 2 == 0:
                    sv_buf, dv_buf, sm = src_v0, dst_v0, sem_l0
                    nxt = (src_v1, dst_v1, sem_l1)
                else:
                    sv_buf, dv_buf, sm = src_v1, dst_v1, sem_l1
                    nxt = (src_v0, dst_v0, sem_l0)
                if si + 1 < n_sect:
                    start_load(si + 1, *nxt)
                wait_load(sv_buf, dv_buf, sm)

                def scan_body(g, cnt, sv_buf=sv_buf, dv_buf=dv_buf):
                    sv = sv_buf[pl.ds(g * LANES, LANES)]
                    dv = dv_buf[pl.ds(g * LANES, LANES)]
                    dl = dv - lo
                    m = (dl >= 0) & (dl < CH)
                    plsc.store_compressed(sel_s.at[pl.ds(cnt, LANES)], sv, mask=m)
                    plsc.store_compressed(sel_d.at[pl.ds(cnt, LANES)], dl, mask=m)
                    return cnt + plsc.all_reduce_population_count(m)[0]

                cnt = lax.fori_loop(0, SECT // LANES, scan_body, cnt)

                nfull = cnt // B
                flush_pipelined(nfull)
                base = nfull * B
                for kk in range(B // LANES):
                    tv = sel_s[pl.ds(base + kk * LANES, LANES)]
                    uv = sel_d[pl.ds(base + kk * LANES, LANES)]
                    sel_s[pl.ds(kk * LANES, LANES)] = tv
                    sel_d[pl.ds(kk * LANES, LANES)] = uv
                cnt = cnt - base

            nb = (cnt + (B - 1)) // B
            limit = nb * B
            for kk in range(B // LANES + 1):
                idx = cnt + kk * LANES + lane
                pm = idx < limit
                plsc.store_scatter(sel_s, [idx], jnp.zeros((LANES,), jnp.int32), mask=pm)
                plsc.store_scatter(sel_d, [idx], jnp.full((LANES,), CH, jnp.int32), mask=pm)
            flush_pipelined(nb)
            plsc.subcore_barrier()

            r0 = s * rows_per_sub
            pltpu.sync_copy(acc.at[pl.ds(r0, rows_per_sub)],
                            agg_hbm.at[pl.ds(lo + r0, rows_per_sub)])
            plsc.subcore_barrier()
            return 0

        lax.fori_loop(0, NCHUNK_PER_SC, do_chunk, 0)

    return k(x, src, dst)


def _tc_stats(scale, x, agg, w, b2):
    n, d = x.shape
    tm = 1000

    def body(scale_ref, x_ref, agg_ref, w_ref, b_ref, st_ref):
        i = pl.program_id(0)
        z = scale_ref[0, 0] * x_ref[...] + agg_ref[...]
        h = lax.dot_general(z, w_ref[...], (((1,), (1,)), ((), ())),
                            preferred_element_type=jnp.float32) + b_ref[...]

        @pl.when(i == 0)
        def _():
            st_ref[...] = jnp.zeros_like(st_ref)

        st_ref[0:1, :] += jnp.sum(h, axis=0, keepdims=True)
        st_ref[1:2, :] += jnp.sum(h * h, axis=0, keepdims=True)

    return pl.pallas_call(
        body,
        grid=(n // tm,),
        in_specs=[
            pl.BlockSpec(memory_space=pltpu.SMEM),
            pl.BlockSpec((tm, d), lambda i: (i, 0)),
            pl.BlockSpec((tm, d), lambda i: (i, 0)),
            pl.BlockSpec((d, d), lambda i: (0, 0)),
            pl.BlockSpec((1, d), lambda i: (0, 0)),
        ],
        out_specs=pl.BlockSpec((8, d), lambda i: (0, 0)),
        out_shape=jax.ShapeDtypeStruct((8, d), jnp.float32),
    )(scale, x, agg, w, b2)


def _tc_out(scale, x, agg, w, b2, stats, g2, bt2):
    n, d = x.shape
    tm = 1000
    inv_n = 1.0 / n

    def body(scale_ref, x_ref, agg_ref, w_ref, b_ref, st_ref, g_ref, bt_ref,
             o_ref):
        z = scale_ref[0, 0] * x_ref[...] + agg_ref[...]
        h = lax.dot_general(z, w_ref[...], (((1,), (1,)), ((), ())),
                            preferred_element_type=jnp.float32) + b_ref[...]
        mean = st_ref[0:1, :] * inv_n
        var = st_ref[1:2, :] * inv_n - mean * mean
        inv = g_ref[...] * lax.rsqrt(var + 1e-5)
        shift = bt_ref[...] - mean * inv
        o_ref[...] = jnp.maximum(h * inv + shift, 0.0) + x_ref[...]

    return pl.pallas_call(
        body,
        grid=(n // tm,),
        in_specs=[
            pl.BlockSpec(memory_space=pltpu.SMEM),
            pl.BlockSpec((tm, d), lambda i: (i, 0)),
            pl.BlockSpec((tm, d), lambda i: (i, 0)),
            pl.BlockSpec((d, d), lambda i: (0, 0)),
            pl.BlockSpec((1, d), lambda i: (0, 0)),
            pl.BlockSpec((8, d), lambda i: (0, 0)),
            pl.BlockSpec((1, d), lambda i: (0, 0)),
            pl.BlockSpec((1, d), lambda i: (0, 0)),
        ],
        out_specs=pl.BlockSpec((tm, d), lambda i: (i, 0)),
        out_shape=jax.ShapeDtypeStruct((n, d), jnp.float32),
    )(scale, x, agg, w, b2, stats, g2, bt2)


def kernel(x, edge_index, batch, W, b, eps, gamma, beta):
    n, d = x.shape
    src = edge_index[0]
    dst = edge_index[1]
    agg = _sc_agg(x, src, dst)
    scale = (1.0 + eps).astype(jnp.float32).reshape(1, 1)
    b2 = b.reshape(1, d)
    stats = _tc_stats(scale, x, agg, W, b2)
    return _tc_out(scale, x, agg, W, b2, stats, gamma.reshape(1, d),
                   beta.reshape(1, d))
